# baseline (speedup 1.0000x reference)
import jax
import jax.numpy as jnp
from jax.experimental import pallas as pl

_H = 4
_HD = 64
_N = 256


def _block_diag_scores(a):
    H, HD = a.shape
    eye = jnp.eye(H, dtype=a.dtype)
    z = a[:, :, None] * eye[:, None, :]
    return z.reshape(H * HD, H)


def _gat_mpnn_kernel(x_ref, w0_ref, ss0_ref, sd0_ref, b0_ref,
                     w1_ref, ss1_ref, sd1_ref, b1_ref,
                     w2_ref, ss2_ref, sd2_ref, b2_ref,
                     wr_ref, br_ref, out_ref):
    f32 = jnp.float32
    B = out_ref.shape[0]
    x = x_ref[...].reshape(B * _N, x_ref.shape[2])
    layers = ((w0_ref, ss0_ref, sd0_ref, b0_ref),
              (w1_ref, ss1_ref, sd1_ref, b1_ref),
              (w2_ref, ss2_ref, sd2_ref, b2_ref))
    ones_col = jnp.ones((_N, 1), dtype=f32)
    for (w_ref, ss_ref, sd_ref, b_ref) in layers:
        xp = jnp.dot(x, w_ref[...], preferred_element_type=f32)
        s = jnp.dot(xp, ss_ref[...], preferred_element_type=f32)
        d = jnp.dot(xp, sd_ref[...], preferred_element_type=f32)
        st = s.T
        outs = []
        for smp in range(B):
            r0 = smp * _N
            smax = jnp.max(s[r0:r0 + _N, :], axis=0, keepdims=True)
            acc = None
            for h in range(_H):
                dh = d[r0:r0 + _N, h:h + 1]
                m = dh + st[h:h + 1, r0:r0 + _N]
                m = jnp.where(m >= 0, m, 0.2 * m)
                rmax = dh + smax[:, h:h + 1]
                rmax = jnp.where(rmax >= 0, rmax, 0.2 * rmax)
                e = jnp.exp(m - rmax)
                xph = jnp.concatenate(
                    [xp[r0:r0 + _N, h * _HD:(h + 1) * _HD], ones_col],
                    axis=1)
                g = jnp.dot(e, xph, preferred_element_type=f32)
                oh = g[:, :_HD] * (1.0 / g[:, _HD:_HD + 1])
                acc = oh if acc is None else acc + oh
            outs.append(jnp.maximum(acc * (1.0 / _H) + b_ref[...], 0.0))
        x = jnp.concatenate(outs, axis=0)
    pooled = jnp.concatenate(
        [jnp.mean(x[smp * _N:(smp + 1) * _N, :], axis=0, keepdims=True)
         for smp in range(B)], axis=0)
    out_ref[...] = jnp.dot(pooled, wr_ref[...], preferred_element_type=f32) + br_ref[...]


def kernel(embeddings, W0, as0, ad0, b0, W1, as1, ad1, b1, W2, as2, ad2, b2, Wr, br):
    B, N, D = embeddings.shape
    ss0, sd0 = _block_diag_scores(as0), _block_diag_scores(ad0)
    ss1, sd1 = _block_diag_scores(as1), _block_diag_scores(ad1)
    ss2, sd2 = _block_diag_scores(as2), _block_diag_scores(ad2)
    b0r, b1r, b2r, brr = (b0.reshape(1, -1), b1.reshape(1, -1),
                          b2.reshape(1, -1), br.reshape(1, -1))
    Dout = Wr.shape[1]

    full = lambda arr: pl.BlockSpec(arr.shape, lambda: (0,) * arr.ndim)
    ins = [embeddings, W0, ss0, sd0, b0r, W1, ss1, sd1, b1r, W2, ss2, sd2, b2r, Wr, brr]
    in_specs = [full(a) for a in ins]

    out = pl.pallas_call(
        _gat_mpnn_kernel,
        in_specs=in_specs,
        out_specs=pl.BlockSpec((B, Dout), lambda: (0, 0)),
        out_shape=jax.ShapeDtypeStruct((B, Dout), jnp.float32),
    )(*ins)
    return out


# speedup vs baseline: 1.2080x; 1.2080x over previous
import jax
import jax.numpy as jnp
from jax.experimental import pallas as pl
from jax.experimental.pallas import tpu as pltpu

_H = 4
_HD = 64
_N = 256


def _block_diag_scores(a):
    H, HD = a.shape
    eye = jnp.eye(H, dtype=a.dtype)
    z = a[:, :, None] * eye[:, None, :]
    return z.reshape(H * HD, H)


def _gat_mpnn_kernel(x_ref, w0_ref, ss0_ref, sd0_ref, b0_ref,
                     w1_ref, ss1_ref, sd1_ref, b1_ref,
                     w2_ref, ss2_ref, sd2_ref, b2_ref,
                     wr_ref, br_ref, out_ref):
    f32 = jnp.float32
    x = x_ref[0]
    layers = ((w0_ref, ss0_ref, sd0_ref, b0_ref),
              (w1_ref, ss1_ref, sd1_ref, b1_ref),
              (w2_ref, ss2_ref, sd2_ref, b2_ref))
    ones_col = jnp.ones((_N, 1), dtype=f32)
    for (w_ref, ss_ref, sd_ref, b_ref) in layers:
        xp = jnp.dot(x, w_ref[...], preferred_element_type=f32)
        s = jnp.dot(xp, ss_ref[...], preferred_element_type=f32)
        d = jnp.dot(xp, sd_ref[...], preferred_element_type=f32)
        st = s.T
        smax = jnp.max(s, axis=0, keepdims=True)
        acc = None
        for h in range(_H):
            dh = d[:, h:h + 1]
            m = dh + st[h:h + 1, :]
            m = jnp.where(m >= 0, m, 0.2 * m)
            rmax = dh + smax[:, h:h + 1]
            rmax = jnp.where(rmax >= 0, rmax, 0.2 * rmax)
            e = jnp.exp(m - rmax)
            xph = jnp.concatenate(
                [xp[:, h * _HD:(h + 1) * _HD], ones_col], axis=1)
            g = jnp.dot(e, xph, preferred_element_type=f32)
            oh = g[:, :_HD] * (1.0 / g[:, _HD:_HD + 1])
            acc = oh if acc is None else acc + oh
        x = jnp.maximum(acc * (1.0 / _H) + b_ref[...], 0.0)
    pooled = jnp.mean(x, axis=0, keepdims=True)
    out_ref[0] = jnp.dot(pooled, wr_ref[...], preferred_element_type=f32) + br_ref[...]


def kernel(embeddings, W0, as0, ad0, b0, W1, as1, ad1, b1, W2, as2, ad2, b2, Wr, br):
    B, N, D = embeddings.shape
    ss0, sd0 = _block_diag_scores(as0), _block_diag_scores(ad0)
    ss1, sd1 = _block_diag_scores(as1), _block_diag_scores(ad1)
    ss2, sd2 = _block_diag_scores(as2), _block_diag_scores(ad2)
    b0r, b1r, b2r, brr = (b0.reshape(1, -1), b1.reshape(1, -1),
                          b2.reshape(1, -1), br.reshape(1, -1))
    Dout = Wr.shape[1]

    full = lambda arr: pl.BlockSpec(arr.shape, lambda b: (0,) * arr.ndim)
    ins = [embeddings, W0, ss0, sd0, b0r, W1, ss1, sd1, b1r, W2, ss2, sd2, b2r, Wr, brr]
    in_specs = [pl.BlockSpec((1, N, D), lambda b: (b, 0, 0))] + [full(a) for a in ins[1:]]

    out = pl.pallas_call(
        _gat_mpnn_kernel,
        grid=(B,),
        in_specs=in_specs,
        out_specs=pl.BlockSpec((1, 1, Dout), lambda b: (b, 0, 0)),
        out_shape=jax.ShapeDtypeStruct((B, 1, Dout), jnp.float32),
        compiler_params=pltpu.CompilerParams(
            dimension_semantics=("parallel",)),
    )(*ins)
    return out.reshape(B, Dout)
